# baseline (speedup 1.0000x reference)
import functools

import jax
import jax.numpy as jnp
from jax import lax
from jax.experimental import pallas as pl
from jax.experimental.pallas import tpu as pltpu
from jax.experimental.pallas import tpu_sc as plsc

N_NODE = 10000
D = 128
DE = 16
E = 160000

NC = 2
NS = 16
NW = NC * NS
K = 128
CHUNKS = E // K
ECEIL = -(-CHUNKS // NW)
DS = 32

_i32 = jnp.int32
_f32 = jnp.float32


def _sc_body(x_hbm, src_hbm, m_out, sidx, rows, sem):
  c = lax.axis_index("c")
  t = lax.axis_index("s")
  w = c * NS + t

  def chunk_body(k, _):
    ci = w + NW * k

    @pl.when(ci < CHUNKS)
    def _():
      off = ci * K
      pltpu.sync_copy(src_hbm.at[pl.ds(off, K)], sidx.at[0])
      pltpu.async_copy(x_hbm.at[sidx.at[0]], rows, sem).wait()
      pltpu.sync_copy(rows, m_out.at[pl.ds(off, K)])
    return 0

  lax.fori_loop(0, ECEIL, chunk_body, 0)


_sc_gather = functools.partial(
    pl.kernel,
    out_type=[jax.ShapeDtypeStruct((E, D), _f32)],
    mesh=plsc.VectorSubcoreMesh(core_axis_name="c", subcore_axis_name="s"),
    scratch_types=[
        pltpu.VMEM((1, K), _i32),
        pltpu.VMEM((K, D), _f32),
        pltpu.SemaphoreType.DMA,
    ],
)(_sc_body)


_R = 1000


def _combine_body(acc_ref, sea_ref, w_ref, wa_ref, out_ref):
  h = jnp.dot(acc_ref[...], w_ref[...], preferred_element_type=_f32)
  h = h + jnp.dot(sea_ref[...], wa_ref[...], preferred_element_type=_f32)
  cnt = sea_ref[:, DE:DE + 1]
  out_ref[...] = h / jnp.maximum(cnt, 1.0)


def _combine(acc, sea_aug, w, w_aug):
  return pl.pallas_call(
      _combine_body,
      grid=(N_NODE // _R,),
      in_specs=[
          pl.BlockSpec((_R, D), lambda i: (i, 0)),
          pl.BlockSpec((_R, DS), lambda i: (i, 0)),
          pl.BlockSpec((D, D), lambda i: (0, 0)),
          pl.BlockSpec((DS, D), lambda i: (0, 0)),
      ],
      out_specs=pl.BlockSpec((_R, D), lambda i: (i, 0)),
      out_shape=jax.ShapeDtypeStruct((N_NODE, D), _f32),
  )(acc, sea_aug, w, w_aug)


def _augment_w(w_edge, b):
  return jnp.concatenate(
      [w_edge, b.reshape(1, D), jnp.zeros((DS - DE - 1, D), _f32)], axis=0)


def _etype(x_src, src, dst, ea, w_src, w_edge, b):
  (m_rows,) = _sc_gather(x_src, src)
  seg = jax.ops.segment_sum(m_rows, dst, num_segments=N_NODE)
  ea_aug = jnp.concatenate(
      [ea, jnp.ones((E, 1), _f32), jnp.zeros((E, DS - DE - 1), _f32)], axis=1)
  seg_aug = jax.ops.segment_sum(ea_aug, dst, num_segments=N_NODE)
  return _combine(seg, seg_aug, w_src, _augment_w(w_edge, b))


def kernel(x_user, x_item, ei_ui, ea_ui, ei_iu, ea_iu,
           W_src_ui, W_edge_ui, b_ui, W_src_iu, W_edge_iu, b_iu):
  s_ui = ei_ui[0].astype(_i32)
  d_ui = ei_ui[1].astype(_i32)
  s_iu = ei_iu[0].astype(_i32)
  d_iu = ei_iu[1].astype(_i32)
  h_item = _etype(x_user, s_ui, d_ui, ea_ui, W_src_ui, W_edge_ui, b_ui)
  h_user = _etype(x_item, s_iu, d_iu, ea_iu, W_src_iu, W_edge_iu, b_iu)
  return (h_user, h_item)
